# baseline (speedup 1.0000x reference)
import functools

import jax
import jax.numpy as jnp
from jax import lax
from jax.experimental import pallas as pl
from jax.experimental.pallas import tpu as pltpu
from jax.experimental.pallas import tpu_sc as plsc

_B = 16384
_D = 64
_CHUNK = 128


def _make_sc_gather():
    info = plsc.get_sparse_core_info()
    nc, ns = info.num_cores, info.num_subcores
    nw = nc * ns
    b_per_w = _B // nw
    n_chunks = b_per_w // _CHUNK

    mesh = plsc.VectorSubcoreMesh(core_axis_name="c", subcore_axis_name="s")
    row_t = jax.ShapeDtypeStruct((_B, _D), jnp.float32)

    @functools.partial(
        pl.kernel,
        mesh=mesh,
        compiler_params=pltpu.CompilerParams(use_tc_tiling_on_sc=False),
        out_type=[row_t, row_t, row_t, row_t],
        scratch_types=[
            pltpu.VMEM((n_chunks, _CHUNK), jnp.int32),
            pltpu.VMEM((n_chunks, _CHUNK), jnp.int32),
            pltpu.VMEM((b_per_w, _D), jnp.float32),
            pltpu.VMEM((b_per_w, _D), jnp.float32),
            pltpu.SemaphoreType.DMA,
        ],
    )
    def sc_gather(uidx_hbm, iidx_hbm, ueg, ieg, uem, iem,
                  out_ug, out_ig, out_um, out_im,
                  uidx_v, iidx_v, buf_a, buf_b, sem):
        wid = lax.axis_index("s") * nc + lax.axis_index("c")
        base = wid * b_per_w
        crow = wid * n_chunks
        pltpu.sync_copy(uidx_hbm.at[pl.ds(crow, n_chunks)], uidx_v)
        pltpu.sync_copy(iidx_hbm.at[pl.ds(crow, n_chunks)], iidx_v)

        def gather_pair(tab_u, tab_i):
            cps = []
            for j in range(n_chunks):
                cps.append(pltpu.async_copy(
                    tab_u.at[uidx_v.at[j]],
                    buf_a.at[pl.ds(j * _CHUNK, _CHUNK)], sem))
                cps.append(pltpu.async_copy(
                    tab_i.at[iidx_v.at[j]],
                    buf_b.at[pl.ds(j * _CHUNK, _CHUNK)], sem))
            for c in cps:
                c.wait()

        gather_pair(ueg, ieg)
        pltpu.sync_copy(buf_a, out_ug.at[pl.ds(base, b_per_w)])
        pltpu.sync_copy(buf_b, out_ig.at[pl.ds(base, b_per_w)])
        gather_pair(uem, iem)
        pltpu.sync_copy(buf_a, out_um.at[pl.ds(base, b_per_w)])
        pltpu.sync_copy(buf_b, out_im.at[pl.ds(base, b_per_w)])

    return sc_gather


_sc_gather = _make_sc_gather()

_BLK = 2048


def _mlp_body(ug, ig, um, im, w1u, w1i, w2, w3, wog, woh,
              b1, b2, b3, bo, out):
    f32 = jnp.float32
    gmf = ug[...] * ig[...]
    h = jnp.dot(um[...], w1u[...], preferred_element_type=f32)
    h = h + jnp.dot(im[...], w1i[...], preferred_element_type=f32)
    h = jnp.maximum(h + b1[...], 0.0)
    h = jnp.maximum(jnp.dot(h, w2[...], preferred_element_type=f32) + b2[...], 0.0)
    h = jnp.maximum(jnp.dot(h, w3[...], preferred_element_type=f32) + b3[...], 0.0)
    o = jnp.dot(gmf, wog[...], preferred_element_type=f32)
    o = o + jnp.dot(h, woh[...], preferred_element_type=f32)
    out[...] = o[:, 0] + bo[0, 0]


def _mlp(ug, ig, um, im, w1u, w1i, w2t, w3t, wog, woh, b1, b2, b3, bo):
    grid = _B // _BLK
    row_spec = pl.BlockSpec((_BLK, _D), lambda i: (i, 0))
    full = lambda a: pl.BlockSpec(a.shape, lambda i: tuple(0 for _ in a.shape))
    return pl.pallas_call(
        _mlp_body,
        grid=(grid,),
        in_specs=[row_spec, row_spec, row_spec, row_spec,
                  full(w1u), full(w1i), full(w2t), full(w3t),
                  full(wog), full(woh),
                  full(b1), full(b2), full(b3), full(bo)],
        out_specs=pl.BlockSpec((_BLK,), lambda i: (i,)),
        out_shape=jax.ShapeDtypeStruct((_B,), jnp.float32),
    )(ug, ig, um, im, w1u, w1i, w2t, w3t, wog, woh, b1, b2, b3, bo)


def kernel(user_indices, item_indices, user_emb_gmf, item_emb_gmf,
           user_emb_mlp, item_emb_mlp, W1, b1, W2, b2, W3, b3, Wo, bo):
    uidx = user_indices.astype(jnp.int32).reshape(_B // _CHUNK, _CHUNK)
    iidx = item_indices.astype(jnp.int32).reshape(_B // _CHUNK, _CHUNK)
    ug, ig, um, im = _sc_gather(uidx, iidx, user_emb_gmf, item_emb_gmf,
                                user_emb_mlp, item_emb_mlp)
    w1t = W1.T
    wot = Wo.T
    return _mlp(ug, ig, um, im,
                w1t[:_D], w1t[_D:], W2.T, W3.T,
                wot[:_D], wot[_D:],
                b1.reshape(1, -1), b2.reshape(1, -1), b3.reshape(1, -1),
                bo.reshape(1, 1))


# speedup vs baseline: 3.6997x; 3.6997x over previous
import functools

import jax
import jax.numpy as jnp
from jax import lax
from jax.experimental import pallas as pl
from jax.experimental.pallas import tpu as pltpu
from jax.experimental.pallas import tpu_sc as plsc

_B = 16384
_D = 64
_V = 1000000
_RW = 5376
_NBF = 186
_EDGE0 = 999936
_EDGEW = 64
_NG = _B // 16
_RING = 2048
_NV = 1280

_info = plsc.get_sparse_core_info()
_NC, _NS = _info.num_cores, _info.num_subcores
_MESH = dict(core_axis_name="c", subcore_axis_name="s")


def _build_worklist(sv):
    i32 = jnp.int32
    jb = sv[0::16] // _RW
    je = sv[15::16] // _RW
    span = je - jb + 1
    starts = jnp.cumsum(span) - span
    vis = jnp.arange(_NV, dtype=i32)
    gvis = jnp.cumsum(jnp.zeros((_NV,), i32).at[starts].add(1)) - 1
    gvis = jnp.clip(gvis, 0, _NG - 1)
    off = vis - starts[gvis]
    jvis = jnp.minimum(jb[gvis] + off, je[gvis])
    jprev = jnp.concatenate([jnp.full((1,), -1, i32), jvis[:-1]])
    nb = (jvis != jprev)
    dseq = jnp.cumsum(nb.astype(i32)) - 1
    bb = dseq % 2
    dblocks = jnp.zeros((_NV + 1,), i32).at[dseq].set(jvis)
    jnext = dblocks[jnp.minimum(dseq + 1, _NV)]
    ndist = dseq[-1] + 1
    ji = (dseq + 1) < ndist
    last = jnp.concatenate([gvis[1:] != gvis[:-1], jnp.full((1,), True)])
    fl = last & ((gvis + 1) % 8 == 0) & (gvis >= 15)
    fw = fl & ((gvis + 1) >= 80)
    return (jvis | (jnext << 8) | (gvis << 16)
            | (nb.astype(i32) << 26) | (ji.astype(i32) << 27)
            | (bb << 28) | (fl.astype(i32) << 29) | (fw.astype(i32) << 30))


def _make_phase_a():
    mesh = plsc.VectorSubcoreMesh(**_MESH)
    out_t = jax.ShapeDtypeStruct((_D, _B), jnp.float32)
    i32 = jnp.int32

    @functools.partial(
        pl.kernel,
        mesh=mesh,
        compiler_params=pltpu.CompilerParams(needs_layout_passes=False),
        out_type=[out_t, out_t, out_t, out_t],
        scratch_types=[
            pltpu.VMEM((_B,), i32),
            pltpu.VMEM((_NV,), i32),
            pltpu.VMEM((2, 8, _RW), jnp.float32),
            pltpu.VMEM((8, _RING), jnp.float32),
            pltpu.SemaphoreType.DMA,
            pltpu.SemaphoreType.DMA,
        ],
    )
    def phase_a(su, si, wlu, wli, t0, t1, t2, t3, e0, e1, e2, e3,
                o0, o1, o2, o3,
                sidx, wl_v, buf, stg, sem_s, sem_f):
        wid = lax.axis_index("s") * _NC + lax.axis_index("c")
        t = wid // 8
        a = wid % 8
        a8 = pl.multiple_of(a * 8, 8)
        lidx = lax.broadcasted_iota(i32, (16,), 0)
        zero16 = jnp.zeros((16,), i32)
        neg = jnp.int32(-2147483648)

        def extract(vec, lane):
            return jnp.max(jnp.where(lidx == lane, vec, neg))

        def run(tab, tail, out, idx_hbm, wl_hbm):
            slab = tab.at[pl.ds(a, 1)]
            pltpu.sync_copy(idx_hbm, sidx)
            pltpu.sync_copy(wl_hbm, wl_v)

            def full_cp(j, b):
                src = slab.at[:, :, pl.ds(pl.multiple_of(j * _RW, 128), _RW)]
                return src, buf.at[pl.ds(b, 1)]

            def edge_cp(j, b):
                src = tail.at[pl.ds(a, 1)]
                return src, buf.at[pl.ds(b, 1)].at[:, :, pl.ds(0, 128)]

            w0 = extract(wl_v[pl.ds(0, 16)], 0)
            j0 = w0 & 0xFF

            @pl.when(j0 < _NBF)
            def _():
                pltpu.async_copy(*full_cp(j0, 0), sem_s)

            @pl.when(j0 == _NBF)
            def _():
                pltpu.async_copy(*edge_cp(j0, 0), sem_s)

            def body(vi, acc):
                v16 = wl_v[pl.ds((vi // 16) * 16, 16)]
                w = extract(v16, lax.rem(vi, 16))
                j = w & 0xFF
                jnx = (w >> 8) & 0xFF
                g = (w >> 16) & 0x3FF
                nb = (w >> 26) & 1
                ji = (w >> 27) & 1
                bb = (w >> 28) & 1
                fl = (w >> 29) & 1
                fw = (w >> 30) & 1

                issue = (nb == 1) & (ji == 1)

                @pl.when(issue & (jnx < _NBF))
                def _():
                    pltpu.async_copy(*full_cp(jnx, 1 - bb), sem_s)

                @pl.when(issue & (jnx == _NBF))
                def _():
                    pltpu.async_copy(*edge_cp(jnx, 1 - bb), sem_s)

                @pl.when((nb == 1) & (j < _NBF))
                def _():
                    pltpu.make_async_copy(*full_cp(j, bb), sem_s).wait()

                @pl.when((nb == 1) & (j == _NBF))
                def _():
                    pltpu.make_async_copy(*edge_cp(j, bb), sem_s).wait()

                r0 = j * _RW
                rend = jnp.where(j == _NBF, jnp.int32(_V), r0 + _RW)
                v = sidx[pl.ds(g * 16, 16)]
                m = jnp.logical_and(v >= r0, v < rend)
                rr = v - r0
                cur = buf.at[pl.ds(bb, 1)]
                cols = lax.rem(g * 16, _RING) + lidx
                for s in range(8):
                    s16 = jnp.full((16,), s, i32)
                    val = plsc.load_gather(cur, [zero16, s16, rr], mask=m)
                    plsc.store_scatter(stg, [s16, cols], val, mask=m)

                @pl.when(fl == 1)
                def _():
                    cstart = pl.multiple_of((g - 15) * 16, 128)
                    ring_c = pl.multiple_of(lax.rem(cstart, _RING), 128)
                    pltpu.async_copy(
                        stg.at[:, pl.ds(ring_c, 128)],
                        out.at[pl.ds(a8, 8), pl.ds(cstart, 128)], sem_f)

                @pl.when(fw == 1)
                def _():
                    pltpu.make_async_copy(
                        stg.at[:, pl.ds(0, 128)],
                        out.at[pl.ds(0, 8), pl.ds(0, 128)], sem_f).wait()

                return acc

            lax.fori_loop(0, _NV, body, jnp.int32(0))
            pltpu.async_copy(
                stg.at[:, pl.ds(_RING - 128, 128)],
                out.at[pl.ds(a8, 8), pl.ds(_B - 128, 128)], sem_f)
            for _ in range(9):
                pltpu.make_async_copy(
                    stg.at[:, pl.ds(0, 128)],
                    out.at[pl.ds(0, 8), pl.ds(0, 128)], sem_f).wait()

        pairs = ((t0, e0, o0, su, wlu), (t1, e1, o1, si, wli),
                 (t2, e2, o2, su, wlu), (t3, e3, o3, si, wli))
        for k, (tb, eb, ob, ih, wh) in enumerate(pairs):
            @pl.when(t == k)
            def _(tb=tb, eb=eb, ob=ob, ih=ih, wh=wh):
                run(tb, eb, ob, ih, wh)

    return phase_a


def _make_phase_b():
    b_per_w = _B // (_NC * _NS)
    n_chunks = b_per_w // 128
    mesh = plsc.VectorSubcoreMesh(**_MESH)
    row_t = jax.ShapeDtypeStruct((_B, _D), jnp.float32)

    @functools.partial(
        pl.kernel,
        mesh=mesh,
        compiler_params=pltpu.CompilerParams(use_tc_tiling_on_sc=False),
        out_type=[row_t, row_t, row_t, row_t],
        scratch_types=[
            pltpu.VMEM((n_chunks, 128), jnp.int32),
            pltpu.VMEM((n_chunks, 128), jnp.int32),
            pltpu.VMEM((b_per_w, _D), jnp.float32),
            pltpu.VMEM((b_per_w, _D), jnp.float32),
            pltpu.SemaphoreType.DMA,
        ],
    )
    def phase_b(ur_hbm, ir_hbm, s0, s1, s2, s3,
                out_ug, out_ig, out_um, out_im,
                uidx_v, iidx_v, buf_a, buf_b, sem):
        wid = lax.axis_index("s") * _NC + lax.axis_index("c")
        base = wid * b_per_w
        crow = wid * n_chunks
        pltpu.sync_copy(ur_hbm.at[pl.ds(crow, n_chunks)], uidx_v)
        pltpu.sync_copy(ir_hbm.at[pl.ds(crow, n_chunks)], iidx_v)

        def gather_pair(tab_u, tab_i):
            cps = []
            for j in range(n_chunks):
                cps.append(pltpu.async_copy(
                    tab_u.at[uidx_v.at[j]],
                    buf_a.at[pl.ds(j * 128, 128)], sem))
                cps.append(pltpu.async_copy(
                    tab_i.at[iidx_v.at[j]],
                    buf_b.at[pl.ds(j * 128, 128)], sem))
            for c in cps:
                c.wait()

        gather_pair(s0, s1)
        pltpu.sync_copy(buf_a, out_ug.at[pl.ds(base, b_per_w)])
        pltpu.sync_copy(buf_b, out_ig.at[pl.ds(base, b_per_w)])
        gather_pair(s2, s3)
        pltpu.sync_copy(buf_a, out_um.at[pl.ds(base, b_per_w)])
        pltpu.sync_copy(buf_b, out_im.at[pl.ds(base, b_per_w)])

    return phase_b


_phase_a = _make_phase_a()
_phase_b = _make_phase_b()

_BLK = 2048


def _mlp_body(ug, ig, um, im, w1u, w1i, w2, w3, wog, woh,
              b1, b2, b3, bo, out):
    f32 = jnp.float32
    gmf = ug[...] * ig[...]
    h = jnp.dot(um[...], w1u[...], preferred_element_type=f32)
    h = h + jnp.dot(im[...], w1i[...], preferred_element_type=f32)
    h = jnp.maximum(h + b1[...], 0.0)
    h = jnp.maximum(jnp.dot(h, w2[...], preferred_element_type=f32) + b2[...], 0.0)
    h = jnp.maximum(jnp.dot(h, w3[...], preferred_element_type=f32) + b3[...], 0.0)
    o = jnp.dot(gmf, wog[...], preferred_element_type=f32)
    o = o + jnp.dot(h, woh[...], preferred_element_type=f32)
    out[...] = o[:, 0] + bo[0, 0]


def _mlp(ug, ig, um, im, w1u, w1i, w2t, w3t, wog, woh, b1, b2, b3, bo):
    grid = _B // _BLK
    row_spec = pl.BlockSpec((_BLK, _D), lambda i: (i, 0))
    full = lambda a: pl.BlockSpec(a.shape, lambda i: tuple(0 for _ in a.shape))
    return pl.pallas_call(
        _mlp_body,
        grid=(grid,),
        in_specs=[row_spec, row_spec, row_spec, row_spec,
                  full(w1u), full(w1i), full(w2t), full(w3t),
                  full(wog), full(woh),
                  full(b1), full(b2), full(b3), full(bo)],
        out_specs=pl.BlockSpec((_BLK,), lambda i: (i,)),
        out_shape=jax.ShapeDtypeStruct((_B,), jnp.float32),
    )(ug, ig, um, im, w1u, w1i, w2t, w3t, wog, woh, b1, b2, b3, bo)


def kernel(user_indices, item_indices, user_emb_gmf, item_emb_gmf,
           user_emb_mlp, item_emb_mlp, W1, b1, W2, b2, W3, b3, Wo, bo):
    i32 = jnp.int32
    uidx = user_indices.astype(i32)
    iidx = item_indices.astype(i32)
    order_u = jnp.argsort(uidx)
    order_i = jnp.argsort(iidx)
    su = jnp.take(uidx, order_u)
    si = jnp.take(iidx, order_i)
    iot = jnp.arange(_B, dtype=i32)
    rank_u = jnp.zeros((_B,), i32).at[order_u].set(iot)
    rank_i = jnp.zeros((_B,), i32).at[order_i].set(iot)
    wlu = _build_worklist(su)
    wli = _build_worklist(si)

    view = lambda tt: tt.T.reshape(8, 8, _V)
    tail = lambda tt: jnp.pad(tt.T[:, _EDGE0:],
                              ((0, 0), (0, 128 - _EDGEW))).reshape(8, 8, 128)
    ts0, ts1, ts2, ts3 = _phase_a(su, si, wlu, wli,
                                  view(user_emb_gmf), view(item_emb_gmf),
                                  view(user_emb_mlp), view(item_emb_mlp),
                                  tail(user_emb_gmf), tail(item_emb_gmf),
                                  tail(user_emb_mlp), tail(item_emb_mlp))
    ug, ig, um, im = _phase_b(rank_u.reshape(128, 128),
                              rank_i.reshape(128, 128),
                              ts0.T, ts1.T, ts2.T, ts3.T)

    w1t = W1.T
    wot = Wo.T
    return _mlp(ug, ig, um, im,
                w1t[:_D], w1t[_D:], W2.T, W3.T,
                wot[:_D], wot[_D:],
                b1.reshape(1, -1), b2.reshape(1, -1), b3.reshape(1, -1),
                bo.reshape(1, 1))


# speedup vs baseline: 3.7662x; 1.0180x over previous
import functools

import jax
import jax.numpy as jnp
from jax import lax
from jax.experimental import pallas as pl
from jax.experimental.pallas import tpu as pltpu
from jax.experimental.pallas import tpu_sc as plsc

_B = 16384
_D = 64
_V = 1000000
_RW = 5376
_NBF = 186
_EDGE0 = 999936
_EDGEW = 64
_NG = _B // 16
_RING = 2048
_NV = 1280

_info = plsc.get_sparse_core_info()
_NC, _NS = _info.num_cores, _info.num_subcores
_MESH = dict(core_axis_name="c", subcore_axis_name="s")


def _build_worklist(sv):
    i32 = jnp.int32
    jb = sv[0::16] // _RW
    je = sv[15::16] // _RW
    span = je - jb + 1
    starts = jnp.cumsum(span) - span
    vis = jnp.arange(_NV, dtype=i32)
    gvis = jnp.cumsum(jnp.zeros((_NV,), i32).at[starts].add(1)) - 1
    gvis = jnp.clip(gvis, 0, _NG - 1)
    off = vis - starts[gvis]
    jvis = jnp.minimum(jb[gvis] + off, je[gvis])
    jprev = jnp.concatenate([jnp.full((1,), -1, i32), jvis[:-1]])
    nb = (jvis != jprev)
    dseq = jnp.cumsum(nb.astype(i32)) - 1
    bb = dseq % 2
    dblocks = jnp.zeros((_NV + 1,), i32).at[dseq].set(jvis)
    jnext = dblocks[jnp.minimum(dseq + 1, _NV)]
    ndist = dseq[-1] + 1
    ji = (dseq + 1) < ndist
    last = jnp.concatenate([gvis[1:] != gvis[:-1], jnp.full((1,), True)])
    fl = last & ((gvis + 1) % 8 == 0) & (gvis >= 15)
    fw = fl & ((gvis + 1) >= 80)
    return (jvis | (jnext << 8) | (gvis << 16)
            | (nb.astype(i32) << 26) | (ji.astype(i32) << 27)
            | (bb << 28) | (fl.astype(i32) << 29) | (fw.astype(i32) << 30))


def _make_phase_a():
    mesh = plsc.VectorSubcoreMesh(**_MESH)
    out_t = jax.ShapeDtypeStruct((_D, _B), jnp.float32)
    i32 = jnp.int32

    @functools.partial(
        pl.kernel,
        mesh=mesh,
        compiler_params=pltpu.CompilerParams(needs_layout_passes=False),
        out_type=[out_t, out_t, out_t, out_t],
        scratch_types=[
            pltpu.VMEM((_B,), i32),
            pltpu.VMEM((_NV,), i32),
            pltpu.VMEM((2, 8, _RW), jnp.float32),
            pltpu.VMEM((8, _RING), jnp.float32),
            pltpu.SemaphoreType.DMA,
            pltpu.SemaphoreType.DMA,
        ],
    )
    def phase_a(su, si, wlu, wli, t0, t1, t2, t3, e0, e1, e2, e3,
                o0, o1, o2, o3,
                sidx, wl_v, buf, stg, sem_s, sem_f):
        wid = lax.axis_index("s") * _NC + lax.axis_index("c")
        t = wid // 8
        a = wid % 8
        a8 = pl.multiple_of(a * 8, 8)
        lidx = lax.broadcasted_iota(i32, (16,), 0)
        zero16 = jnp.zeros((16,), i32)
        neg = jnp.int32(-2147483648)

        def extract(vec, lane):
            return jnp.max(jnp.where(lidx == lane, vec, neg))

        def run(tab, tail, out, idx_hbm, wl_hbm):
            slab = tab.at[pl.ds(a, 1)]
            pltpu.sync_copy(idx_hbm, sidx)
            pltpu.sync_copy(wl_hbm, wl_v)

            def full_cp(j, b):
                src = slab.at[:, :, pl.ds(pl.multiple_of(j * _RW, 128), _RW)]
                return src, buf.at[pl.ds(b, 1)]

            def edge_cp(j, b):
                src = tail.at[pl.ds(a, 1)]
                return src, buf.at[pl.ds(b, 1)].at[:, :, pl.ds(0, 128)]

            w0 = extract(wl_v[pl.ds(0, 16)], 0)
            j0 = w0 & 0xFF

            @pl.when(j0 < _NBF)
            def _():
                pltpu.async_copy(*full_cp(j0, 0), sem_s)

            @pl.when(j0 == _NBF)
            def _():
                pltpu.async_copy(*edge_cp(j0, 0), sem_s)

            def body(vi, acc):
                v16 = wl_v[pl.ds((vi // 16) * 16, 16)]
                w = extract(v16, lax.rem(vi, 16))
                j = w & 0xFF
                jnx = (w >> 8) & 0xFF
                g = (w >> 16) & 0x3FF
                nb = (w >> 26) & 1
                ji = (w >> 27) & 1
                bb = (w >> 28) & 1
                fl = (w >> 29) & 1
                fw = (w >> 30) & 1

                issue = (nb == 1) & (ji == 1)

                @pl.when(issue & (jnx < _NBF))
                def _():
                    pltpu.async_copy(*full_cp(jnx, 1 - bb), sem_s)

                @pl.when(issue & (jnx == _NBF))
                def _():
                    pltpu.async_copy(*edge_cp(jnx, 1 - bb), sem_s)

                @pl.when((nb == 1) & (j < _NBF))
                def _():
                    pltpu.make_async_copy(*full_cp(j, bb), sem_s).wait()

                @pl.when((nb == 1) & (j == _NBF))
                def _():
                    pltpu.make_async_copy(*edge_cp(j, bb), sem_s).wait()

                r0 = j * _RW
                rend = jnp.where(j == _NBF, jnp.int32(_V), r0 + _RW)
                v = sidx[pl.ds(g * 16, 16)]
                m = jnp.logical_and(v >= r0, v < rend)
                rr = v - r0
                cur = buf.at[pl.ds(bb, 1)]
                cols = lax.rem(g * 16, _RING) + lidx
                for s in range(8):
                    s16 = jnp.full((16,), s, i32)
                    val = plsc.load_gather(cur, [zero16, s16, rr], mask=m)
                    plsc.store_scatter(stg, [s16, cols], val, mask=m)

                @pl.when(fl == 1)
                def _():
                    cstart = pl.multiple_of((g - 15) * 16, 128)
                    ring_c = pl.multiple_of(lax.rem(cstart, _RING), 128)
                    pltpu.async_copy(
                        stg.at[:, pl.ds(ring_c, 128)],
                        out.at[pl.ds(a8, 8), pl.ds(cstart, 128)], sem_f)

                @pl.when(fw == 1)
                def _():
                    pltpu.make_async_copy(
                        stg.at[:, pl.ds(0, 128)],
                        out.at[pl.ds(0, 8), pl.ds(0, 128)], sem_f).wait()

                return acc

            lax.fori_loop(0, _NV, body, jnp.int32(0))
            pltpu.async_copy(
                stg.at[:, pl.ds(_RING - 128, 128)],
                out.at[pl.ds(a8, 8), pl.ds(_B - 128, 128)], sem_f)
            for _ in range(9):
                pltpu.make_async_copy(
                    stg.at[:, pl.ds(0, 128)],
                    out.at[pl.ds(0, 8), pl.ds(0, 128)], sem_f).wait()

        pairs = ((t0, e0, o0, su, wlu), (t1, e1, o1, si, wli),
                 (t2, e2, o2, su, wlu), (t3, e3, o3, si, wli))
        for k, (tb, eb, ob, ih, wh) in enumerate(pairs):
            @pl.when(t == k)
            def _(tb=tb, eb=eb, ob=ob, ih=ih, wh=wh):
                run(tb, eb, ob, ih, wh)

    return phase_a


def _make_phase_b():
    b_per_w = _B // (_NC * _NS)
    n_chunks = b_per_w // 128
    mesh = plsc.VectorSubcoreMesh(**_MESH)
    row_t = jax.ShapeDtypeStruct((_B, _D), jnp.float32)

    @functools.partial(
        pl.kernel,
        mesh=mesh,
        compiler_params=pltpu.CompilerParams(use_tc_tiling_on_sc=False),
        out_type=[row_t, row_t, row_t, row_t],
        scratch_types=[
            pltpu.VMEM((n_chunks, 128), jnp.int32),
            pltpu.VMEM((n_chunks, 128), jnp.int32),
            pltpu.VMEM((b_per_w, _D), jnp.float32),
            pltpu.VMEM((b_per_w, _D), jnp.float32),
            pltpu.SemaphoreType.DMA,
        ],
    )
    def phase_b(ur_hbm, ir_hbm, s0, s1, s2, s3,
                out_ug, out_ig, out_um, out_im,
                uidx_v, iidx_v, buf_a, buf_b, sem):
        wid = lax.axis_index("s") * _NC + lax.axis_index("c")
        base = wid * b_per_w
        crow = wid * n_chunks
        pltpu.sync_copy(ur_hbm.at[pl.ds(crow, n_chunks)], uidx_v)
        pltpu.sync_copy(ir_hbm.at[pl.ds(crow, n_chunks)], iidx_v)

        def gather_pair(tab_u, tab_i):
            cps = []
            for j in range(n_chunks):
                cps.append(pltpu.async_copy(
                    tab_u.at[uidx_v.at[j]],
                    buf_a.at[pl.ds(j * 128, 128)], sem))
                cps.append(pltpu.async_copy(
                    tab_i.at[iidx_v.at[j]],
                    buf_b.at[pl.ds(j * 128, 128)], sem))
            for c in cps:
                c.wait()

        gather_pair(s0, s1)
        pltpu.sync_copy(buf_a, out_ug.at[pl.ds(base, b_per_w)])
        pltpu.sync_copy(buf_b, out_ig.at[pl.ds(base, b_per_w)])
        gather_pair(s2, s3)
        pltpu.sync_copy(buf_a, out_um.at[pl.ds(base, b_per_w)])
        pltpu.sync_copy(buf_b, out_im.at[pl.ds(base, b_per_w)])

    return phase_b


_phase_a = _make_phase_a()
_phase_b = _make_phase_b()

_BLK = 2048


def _mlp_body(ug, ig, um, im, w1u, w1i, w2, w3, wog, woh,
              b1, b2, b3, bo, out):
    f32 = jnp.float32
    gmf = ug[...] * ig[...]
    h = jnp.dot(um[...], w1u[...], preferred_element_type=f32)
    h = h + jnp.dot(im[...], w1i[...], preferred_element_type=f32)
    h = jnp.maximum(h + b1[...], 0.0)
    h = jnp.maximum(jnp.dot(h, w2[...], preferred_element_type=f32) + b2[...], 0.0)
    h = jnp.maximum(jnp.dot(h, w3[...], preferred_element_type=f32) + b3[...], 0.0)
    o = jnp.dot(gmf, wog[...], preferred_element_type=f32)
    o = o + jnp.dot(h, woh[...], preferred_element_type=f32)
    out[...] = o[:, 0] + bo[0, 0]


def _mlp(ug, ig, um, im, w1u, w1i, w2t, w3t, wog, woh, b1, b2, b3, bo):
    grid = _B // _BLK
    row_spec = pl.BlockSpec((_BLK, _D), lambda i: (i, 0))
    full = lambda a: pl.BlockSpec(a.shape, lambda i: tuple(0 for _ in a.shape))
    return pl.pallas_call(
        _mlp_body,
        grid=(grid,),
        in_specs=[row_spec, row_spec, row_spec, row_spec,
                  full(w1u), full(w1i), full(w2t), full(w3t),
                  full(wog), full(woh),
                  full(b1), full(b2), full(b3), full(bo)],
        out_specs=pl.BlockSpec((_BLK,), lambda i: (i,)),
        out_shape=jax.ShapeDtypeStruct((_B,), jnp.float32),
    )(ug, ig, um, im, w1u, w1i, w2t, w3t, wog, woh, b1, b2, b3, bo)


def kernel(user_indices, item_indices, user_emb_gmf, item_emb_gmf,
           user_emb_mlp, item_emb_mlp, W1, b1, W2, b2, W3, b3, Wo, bo):
    i32 = jnp.int32
    uidx = user_indices.astype(i32)
    iidx = item_indices.astype(i32)
    iot = jnp.arange(_B, dtype=i32)
    su, order_u = lax.sort_key_val(uidx, iot)
    si, order_i = lax.sort_key_val(iidx, iot)
    rank_u = jnp.zeros((_B,), i32).at[order_u].set(iot)
    rank_i = jnp.zeros((_B,), i32).at[order_i].set(iot)
    wlu = _build_worklist(su)
    wli = _build_worklist(si)

    view = lambda tt: tt.T.reshape(8, 8, _V)
    tail = lambda tt: jnp.pad(tt.T[:, _EDGE0:],
                              ((0, 0), (0, 128 - _EDGEW))).reshape(8, 8, 128)
    ts0, ts1, ts2, ts3 = _phase_a(su, si, wlu, wli,
                                  view(user_emb_gmf), view(item_emb_gmf),
                                  view(user_emb_mlp), view(item_emb_mlp),
                                  tail(user_emb_gmf), tail(item_emb_gmf),
                                  tail(user_emb_mlp), tail(item_emb_mlp))
    ug, ig, um, im = _phase_b(rank_u.reshape(128, 128),
                              rank_i.reshape(128, 128),
                              ts0.T, ts1.T, ts2.T, ts3.T)

    w1t = W1.T
    wot = Wo.T
    return _mlp(ug, ig, um, im,
                w1t[:_D], w1t[_D:], W2.T, W3.T,
                wot[:_D], wot[_D:],
                b1.reshape(1, -1), b2.reshape(1, -1), b3.reshape(1, -1),
                bo.reshape(1, 1))
